# baseline (speedup 1.0000x reference)
import functools

import jax
import jax.numpy as jnp
from jax import lax
from jax.experimental import pallas as pl
from jax.experimental.pallas import tpu as pltpu
from jax.experimental.pallas import tpu_sc as plsc

N = 10000
E = 320000
EH = E // 2
NODE_FDIM = 133
EDGE_FDIM = 14
HID = 128

NC = 2
NS = 16
NW = NC * NS
EPW = EH // NW
CHUNK = 40
NCHUNK = EPW // CHUNK
GROUP = 5
NGROUP = NCHUNK // GROUP
N_PAD = 10240
NPT = N_PAD // NS


def _worker_id():
    return lax.axis_index("s") * NC + lax.axis_index("c")


def _gather_body(node_hbm, src_hbm, out_hbm, idx_v, buf_a, buf_b, sem_a,
                 sem_b, sem_s):
    wid = _worker_id()
    base = wid * EPW
    pltpu.sync_copy(src_hbm.at[wid], idx_v)

    def fire(g, buf, sem):
        for j in range(GROUP):
            pltpu.async_copy(node_hbm.at[idx_v.at[g * GROUP + j]],
                             buf.at[pl.ds(j * CHUNK, CHUNK)], sem)

    def drain(g, buf, sem):
        for j in range(GROUP):
            pltpu.make_async_copy(node_hbm.at[idx_v.at[g * GROUP + j]],
                                  buf.at[pl.ds(j * CHUNK, CHUNK)], sem).wait()

    def store(g, buf):
        pltpu.async_copy(
            buf, out_hbm.at[pl.ds(base + g * GROUP * CHUNK, GROUP * CHUNK)],
            sem_s).wait()

    fire(0, buf_a, sem_a)

    def pair(k, carry):
        g0 = 2 * k
        fire(g0 + 1, buf_b, sem_b)
        drain(g0, buf_a, sem_a)
        store(g0, buf_a)
        fire(g0 + 2, buf_a, sem_a)
        drain(g0 + 1, buf_b, sem_b)
        store(g0 + 1, buf_b)
        return carry

    lax.fori_loop(0, (NGROUP - 1) // 2, pair, 0)
    drain(NGROUP - 1, buf_a, sem_a)
    store(NGROUP - 1, buf_a)


@functools.cache
def _sc_gather_kernel():
    mesh = plsc.VectorSubcoreMesh(
        core_axis_name="c", subcore_axis_name="s",
        num_cores=NC, num_subcores=NS)
    return pl.kernel(
        _gather_body,
        out_type=jax.ShapeDtypeStruct((EH, HID), jnp.float32),
        mesh=mesh,
        scratch_types=[
            pltpu.VMEM((NCHUNK, CHUNK), jnp.int32),
            pltpu.VMEM((GROUP * CHUNK, HID), jnp.float32),
            pltpu.VMEM((GROUP * CHUNK, HID), jnp.float32),
            pltpu.SemaphoreType.DMA,
            pltpu.SemaphoreType.DMA,
            pltpu.SemaphoreType.DMA,
        ],
    )


def _scatter_body(edge_hbm, dst_hbm, zeros_hbm, out_hbm, idx_v, buf, agg_sh,
                  sem0, sem1):
    cid = lax.axis_index("c")
    sid = lax.axis_index("s")
    wid = sid * NC + cid
    base = wid * EPW
    pltpu.sync_copy(zeros_hbm.at[pl.ds(sid * NPT, NPT)],
                    agg_sh.at[pl.ds(sid * NPT, NPT)])
    pltpu.sync_copy(dst_hbm.at[wid], idx_v)
    plsc.subcore_barrier()

    def load(c, b, sem):
        return pltpu.async_copy(
            edge_hbm.at[pl.ds(base + c * CHUNK, CHUNK)], buf.at[b], sem)

    def wait_load(c, b, sem):
        pltpu.make_async_copy(
            edge_hbm.at[pl.ds(base + c * CHUNK, CHUNK)], buf.at[b], sem).wait()

    def scat(c, b):
        pltpu.sync_copy(buf.at[b], agg_sh.at[idx_v.at[c]], add=True)

    load(0, 0, sem0)

    def pair(k, carry):
        c0 = 2 * k
        load(c0 + 1, 1, sem1)
        wait_load(c0, 0, sem0)
        scat(c0, 0)
        load(c0 + 2, 0, sem0)
        wait_load(c0 + 1, 1, sem1)
        scat(c0 + 1, 1)
        return carry

    lax.fori_loop(0, (NCHUNK - 1) // 2, pair, 0)
    wait_load(NCHUNK - 1, 0, sem0)
    scat(NCHUNK - 1, 0)
    plsc.subcore_barrier()
    pltpu.sync_copy(agg_sh.at[pl.ds(sid * NPT, NPT)],
                    out_hbm.at[cid, pl.ds(sid * NPT, NPT)])


@functools.cache
def _sc_scatter_kernel():
    mesh = plsc.VectorSubcoreMesh(
        core_axis_name="c", subcore_axis_name="s",
        num_cores=NC, num_subcores=NS)
    return pl.kernel(
        _scatter_body,
        out_type=jax.ShapeDtypeStruct((NC, N_PAD, HID), jnp.float32),
        mesh=mesh,
        scratch_types=[
            pltpu.VMEM((NCHUNK, CHUNK), jnp.int32),
            pltpu.VMEM((2, CHUNK, HID), jnp.float32),
            pltpu.VMEM_SHARED((N_PAD, HID), jnp.float32),
            pltpu.SemaphoreType.DMA,
            pltpu.SemaphoreType.DMA,
        ],
    )


_BN = 2000
_BE = 6400
_GE = EH // _BE


def _node_init_body(xt_ref, w_ref, o_ref):
    o_ref[...] = jax.nn.relu(
        jnp.einsum("kn,kh->nh", xt_ref[...], w_ref[...],
                   preferred_element_type=jnp.float32))


_node_init = pl.pallas_call(
    _node_init_body,
    grid=(1,),
    in_specs=[
        pl.BlockSpec((NODE_FDIM, N), lambda i: (0, 0)),
        pl.BlockSpec((NODE_FDIM, HID), lambda i: (0, 0)),
    ],
    out_specs=pl.BlockSpec((N, HID), lambda i: (0, 0)),
    out_shape=jax.ShapeDtypeStruct((N, HID), jnp.float32),
)


def _edge_init_body(eat_ref, w_ref, o_ref):
    o_ref[...] = jax.nn.relu(
        jnp.einsum("ke,kh->eh", eat_ref[...], w_ref[...],
                   preferred_element_type=jnp.float32))


@functools.cache
def _edge_init(half):
    off = half * _GE
    return pl.pallas_call(
        _edge_init_body,
        grid=(_GE,),
        in_specs=[
            pl.BlockSpec((EDGE_FDIM, _BE), lambda i: (0, i + off)),
            pl.BlockSpec((EDGE_FDIM, HID), lambda i: (0, 0)),
        ],
        out_specs=pl.BlockSpec((_BE, HID), lambda i: (i, 0)),
        out_shape=jax.ShapeDtypeStruct((EH, HID), jnp.float32),
    )


def _node_mult_body(agg_a_ref, agg_b_ref, node_ref, o_ref):
    o_ref[...] = node_ref[...] * (agg_a_ref[0] + agg_a_ref[1]
                                  + agg_b_ref[0] + agg_b_ref[1])


_node_mult = pl.pallas_call(
    _node_mult_body,
    grid=(N // _BN,),
    in_specs=[
        pl.BlockSpec((NC, _BN, HID), lambda i: (0, i, 0)),
        pl.BlockSpec((NC, _BN, HID), lambda i: (0, i, 0)),
        pl.BlockSpec((_BN, HID), lambda i: (i, 0)),
    ],
    out_specs=pl.BlockSpec((_BN, HID), lambda i: (i, 0)),
    out_shape=jax.ShapeDtypeStruct((N, HID), jnp.float32),
)


def _edge_update_body(g_ref, eat_ref, a1_ref, a2_ref, wd_ref, o_ref):
    g16 = g_ref[...].astype(jnp.bfloat16)
    a116 = a1_ref[...].astype(jnp.bfloat16)
    msg = jax.nn.relu(
        jnp.dot(g16, a116, preferred_element_type=jnp.float32)
        + jnp.einsum("ke,kh->eh", eat_ref[...], a2_ref[...],
                     preferred_element_type=jnp.float32))
    o_ref[...] = jax.nn.relu(
        jnp.dot(msg.astype(jnp.bfloat16), wd_ref[...].astype(jnp.bfloat16),
                preferred_element_type=jnp.float32))


@functools.cache
def _edge_update(half):
    off = half * _GE
    return pl.pallas_call(
        _edge_update_body,
        grid=(_GE,),
        in_specs=[
            pl.BlockSpec((_BE, HID), lambda i: (i, 0)),
            pl.BlockSpec((EDGE_FDIM, _BE), lambda i: (0, i + off)),
            pl.BlockSpec((HID, HID), lambda i: (0, 0)),
            pl.BlockSpec((EDGE_FDIM, HID), lambda i: (0, 0)),
            pl.BlockSpec((HID, HID), lambda i: (0, 0)),
        ],
        out_specs=pl.BlockSpec((_BE, HID), lambda i: (i, 0)),
        out_shape=jax.ShapeDtypeStruct((EH, HID), jnp.float32),
    )


def _final_body(agg_a_ref, agg_b_ref, node_ref, orig_ref, l1_ref, l2_ref,
                l3_ref, o1_ref, o2_ref, bo_ref, o_ref):
    agg = agg_a_ref[0] + agg_a_ref[1] + agg_b_ref[0] + agg_b_ref[1]
    n2 = jax.nn.relu(
        jnp.dot(agg, l1_ref[...], preferred_element_type=jnp.float32)
        + jnp.dot(node_ref[...], l2_ref[...], preferred_element_type=jnp.float32)
        + jnp.dot(orig_ref[...], l3_ref[...], preferred_element_type=jnp.float32))
    o_ref[...] = jax.nn.relu(
        jnp.dot(n2, o1_ref[...], preferred_element_type=jnp.float32)
        + jnp.dot(orig_ref[...], o2_ref[...], preferred_element_type=jnp.float32)
        + bo_ref[...])


_final = pl.pallas_call(
    _final_body,
    grid=(N // _BN,),
    in_specs=[
        pl.BlockSpec((NC, _BN, HID), lambda i: (0, i, 0)),
        pl.BlockSpec((NC, _BN, HID), lambda i: (0, i, 0)),
        pl.BlockSpec((_BN, HID), lambda i: (i, 0)),
        pl.BlockSpec((_BN, HID), lambda i: (i, 0)),
        pl.BlockSpec((HID, HID), lambda i: (0, 0)),
        pl.BlockSpec((HID, HID), lambda i: (0, 0)),
        pl.BlockSpec((HID, HID), lambda i: (0, 0)),
        pl.BlockSpec((HID, HID), lambda i: (0, 0)),
        pl.BlockSpec((HID, HID), lambda i: (0, 0)),
        pl.BlockSpec((1, HID), lambda i: (0, 0)),
    ],
    out_specs=pl.BlockSpec((_BN, HID), lambda i: (i, 0)),
    out_shape=jax.ShapeDtypeStruct((N, HID), jnp.float32),
)


def kernel(x, edge_index, edge_attr, W_i_atom, W_i_bond, W_h_atom, W_h_0,
           W_h_1, lr_W, W_o, b_o):
    src_r = edge_index[0].reshape(2, NW, NCHUNK, CHUNK)
    dst_r = edge_index[1].reshape(2, NW, NCHUNK, CHUNK)
    src_h = (src_r[0], src_r[1])
    dst_h = (dst_r[0], dst_r[1])
    zeros = jnp.zeros((N_PAD, HID), jnp.float32)

    xt = x.T
    eat = edge_attr.T
    node_origin = _node_init(xt, W_i_atom.T)
    edge_h = [_edge_init(0)(eat, W_i_bond.T), _edge_init(1)(eat, W_i_bond.T)]

    a1 = W_h_atom[:, :HID].T
    a2 = W_h_atom[:, HID:].T
    node = node_origin
    scatter = _sc_scatter_kernel()
    gather = _sc_gather_kernel()
    for wd in (W_h_0, W_h_1):
        agg_a = scatter(edge_h[0], dst_h[0], zeros)
        agg_b = scatter(edge_h[1], dst_h[1], zeros)
        node = _node_mult(agg_a, agg_b, node)
        gath_a = gather(node, src_h[0])
        gath_b = gather(node, src_h[1])
        wdt = wd.T
        edge_h = [_edge_update(0)(gath_a, eat, a1, a2, wdt),
                  _edge_update(1)(gath_b, eat, a1, a2, wdt)]

    agg_a = scatter(edge_h[0], dst_h[0], zeros)
    agg_b = scatter(edge_h[1], dst_h[1], zeros)
    out = _final(
        agg_a, agg_b, node, node_origin,
        lr_W[:, :HID].T, lr_W[:, HID:2 * HID].T, lr_W[:, 2 * HID:].T,
        W_o[:, :HID].T, W_o[:, HID:].T, b_o.reshape(1, HID))
    return out


# speedup vs baseline: 1.1284x; 1.1284x over previous
import functools

import jax
import jax.numpy as jnp
from jax import lax
from jax.experimental import pallas as pl
from jax.experimental.pallas import tpu as pltpu
from jax.experimental.pallas import tpu_sc as plsc

N = 10000
E = 320000
EA = 166400
EB = E - EA
NODE_FDIM = 133
EDGE_FDIM = 14
HID = 128

NC = 2
NS = 16
NW = NC * NS
CHUNK = 80
GROUP = 5
N_PAD = 10240
NPT = N_PAD // NS


def _worker_id():
    return lax.axis_index("s") * NC + lax.axis_index("c")


def _gather_body(esize, node_hbm, src_hbm, out_hbm, idx_v, buf_a, buf_b,
                 sem_a, sem_b, sem_s):
    epw = esize // NW
    ngroup = epw // (GROUP * CHUNK)
    wid = _worker_id()
    base = wid * epw
    pltpu.sync_copy(src_hbm.at[wid], idx_v)

    def fire(g, buf, sem):
        for j in range(GROUP):
            pltpu.async_copy(node_hbm.at[idx_v.at[g * GROUP + j]],
                             buf.at[pl.ds(j * CHUNK, CHUNK)], sem)

    def drain(g, buf, sem):
        for j in range(GROUP):
            pltpu.make_async_copy(node_hbm.at[idx_v.at[g * GROUP + j]],
                                  buf.at[pl.ds(j * CHUNK, CHUNK)], sem).wait()

    def store(g, buf):
        pltpu.async_copy(
            buf, out_hbm.at[pl.ds(base + g * GROUP * CHUNK, GROUP * CHUNK)],
            sem_s).wait()

    npair = (ngroup - 1) // 2
    fire(0, buf_a, sem_a)

    def pair(k, carry):
        g0 = 2 * k
        fire(g0 + 1, buf_b, sem_b)
        drain(g0, buf_a, sem_a)
        store(g0, buf_a)
        fire(g0 + 2, buf_a, sem_a)
        drain(g0 + 1, buf_b, sem_b)
        store(g0 + 1, buf_b)
        return carry

    lax.fori_loop(0, npair, pair, 0)
    last_a = 2 * npair
    if ngroup % 2 == 0:
        fire(ngroup - 1, buf_b, sem_b)
    drain(last_a, buf_a, sem_a)
    store(last_a, buf_a)
    if ngroup % 2 == 0:
        drain(ngroup - 1, buf_b, sem_b)
        store(ngroup - 1, buf_b)


@functools.cache
def _sc_gather_kernel(esize):
    nchunk = esize // NW // CHUNK
    mesh = plsc.VectorSubcoreMesh(
        core_axis_name="c", subcore_axis_name="s",
        num_cores=NC, num_subcores=NS)
    return pl.kernel(
        functools.partial(_gather_body, esize),
        out_type=jax.ShapeDtypeStruct((esize, HID), jnp.float32),
        mesh=mesh,
        scratch_types=[
            pltpu.VMEM((nchunk, CHUNK), jnp.int32),
            pltpu.VMEM((GROUP * CHUNK, HID), jnp.float32),
            pltpu.VMEM((GROUP * CHUNK, HID), jnp.float32),
            pltpu.SemaphoreType.DMA,
            pltpu.SemaphoreType.DMA,
            pltpu.SemaphoreType.DMA,
        ],
    )


def _scatter_body(esize, edge_hbm, dst_hbm, zeros_hbm, out_hbm, idx_v, buf,
                  agg_sh, sem0, sem1):
    epw = esize // NW
    nchunk = epw // CHUNK
    cid = lax.axis_index("c")
    sid = lax.axis_index("s")
    wid = sid * NC + cid
    base = wid * epw
    pltpu.sync_copy(zeros_hbm.at[pl.ds(sid * NPT, NPT)],
                    agg_sh.at[pl.ds(sid * NPT, NPT)])
    pltpu.sync_copy(dst_hbm.at[wid], idx_v)
    plsc.subcore_barrier()

    def load(c, b, sem):
        return pltpu.async_copy(
            edge_hbm.at[pl.ds(base + c * CHUNK, CHUNK)], buf.at[b], sem)

    def wait_load(c, b, sem):
        pltpu.make_async_copy(
            edge_hbm.at[pl.ds(base + c * CHUNK, CHUNK)], buf.at[b], sem).wait()

    def scat(c, b):
        pltpu.sync_copy(buf.at[b], agg_sh.at[idx_v.at[c]], add=True)

    npair = (nchunk - 1) // 2
    load(0, 0, sem0)

    def pair(k, carry):
        c0 = 2 * k
        load(c0 + 1, 1, sem1)
        wait_load(c0, 0, sem0)
        scat(c0, 0)
        load(c0 + 2, 0, sem0)
        wait_load(c0 + 1, 1, sem1)
        scat(c0 + 1, 1)
        return carry

    lax.fori_loop(0, npair, pair, 0)
    last_a = 2 * npair
    if nchunk % 2 == 0:
        load(nchunk - 1, 1, sem1)
    wait_load(last_a, 0, sem0)
    scat(last_a, 0)
    if nchunk % 2 == 0:
        wait_load(nchunk - 1, 1, sem1)
        scat(nchunk - 1, 1)
    plsc.subcore_barrier()
    pltpu.sync_copy(agg_sh.at[pl.ds(sid * NPT, NPT)],
                    out_hbm.at[cid, pl.ds(sid * NPT, NPT)])


@functools.cache
def _sc_scatter_kernel(esize):
    nchunk = esize // NW // CHUNK
    mesh = plsc.VectorSubcoreMesh(
        core_axis_name="c", subcore_axis_name="s",
        num_cores=NC, num_subcores=NS)
    return pl.kernel(
        functools.partial(_scatter_body, esize),
        out_type=jax.ShapeDtypeStruct((NC, N_PAD, HID), jnp.float32),
        mesh=mesh,
        scratch_types=[
            pltpu.VMEM((nchunk, CHUNK), jnp.int32),
            pltpu.VMEM((2, CHUNK, HID), jnp.float32),
            pltpu.VMEM_SHARED((N_PAD, HID), jnp.float32),
            pltpu.SemaphoreType.DMA,
            pltpu.SemaphoreType.DMA,
        ],
    )


_BN = 2000
_BE = 6400


def _node_init_body(xt_ref, w_ref, o_ref):
    o_ref[...] = jax.nn.relu(
        jnp.einsum("kn,kh->nh", xt_ref[...], w_ref[...],
                   preferred_element_type=jnp.float32))


_node_init = pl.pallas_call(
    _node_init_body,
    grid=(1,),
    in_specs=[
        pl.BlockSpec((NODE_FDIM, N), lambda i: (0, 0)),
        pl.BlockSpec((NODE_FDIM, HID), lambda i: (0, 0)),
    ],
    out_specs=pl.BlockSpec((N, HID), lambda i: (0, 0)),
    out_shape=jax.ShapeDtypeStruct((N, HID), jnp.float32),
)


def _edge_init_body(eat_ref, w_ref, o_ref):
    o_ref[...] = jax.nn.relu(
        jnp.einsum("ke,kh->eh", eat_ref[...], w_ref[...],
                   preferred_element_type=jnp.float32))


@functools.cache
def _edge_init(esize, blk_off):
    return pl.pallas_call(
        _edge_init_body,
        grid=(esize // _BE,),
        in_specs=[
            pl.BlockSpec((EDGE_FDIM, _BE), lambda i: (0, i + blk_off)),
            pl.BlockSpec((EDGE_FDIM, HID), lambda i: (0, 0)),
        ],
        out_specs=pl.BlockSpec((_BE, HID), lambda i: (i, 0)),
        out_shape=jax.ShapeDtypeStruct((esize, HID), jnp.float32),
    )


def _node_mult_body(agg_a_ref, agg_b_ref, node_ref, o_ref):
    o_ref[...] = node_ref[...] * (agg_a_ref[0] + agg_a_ref[1]
                                  + agg_b_ref[0] + agg_b_ref[1])


_node_mult = pl.pallas_call(
    _node_mult_body,
    grid=(N // _BN,),
    in_specs=[
        pl.BlockSpec((NC, _BN, HID), lambda i: (0, i, 0)),
        pl.BlockSpec((NC, _BN, HID), lambda i: (0, i, 0)),
        pl.BlockSpec((_BN, HID), lambda i: (i, 0)),
    ],
    out_specs=pl.BlockSpec((_BN, HID), lambda i: (i, 0)),
    out_shape=jax.ShapeDtypeStruct((N, HID), jnp.float32),
)


def _edge_update_body(g_ref, eat_ref, a1_ref, a2_ref, wd_ref, o_ref):
    g16 = g_ref[...].astype(jnp.bfloat16)
    a116 = a1_ref[...].astype(jnp.bfloat16)
    msg = jax.nn.relu(
        jnp.dot(g16, a116, preferred_element_type=jnp.float32)
        + jnp.einsum("ke,kh->eh", eat_ref[...], a2_ref[...],
                     preferred_element_type=jnp.float32))
    o_ref[...] = jax.nn.relu(
        jnp.dot(msg.astype(jnp.bfloat16), wd_ref[...].astype(jnp.bfloat16),
                preferred_element_type=jnp.float32))


@functools.cache
def _edge_update(esize, blk_off):
    return pl.pallas_call(
        _edge_update_body,
        grid=(esize // _BE,),
        in_specs=[
            pl.BlockSpec((_BE, HID), lambda i: (i, 0)),
            pl.BlockSpec((EDGE_FDIM, _BE), lambda i: (0, i + blk_off)),
            pl.BlockSpec((HID, HID), lambda i: (0, 0)),
            pl.BlockSpec((EDGE_FDIM, HID), lambda i: (0, 0)),
            pl.BlockSpec((HID, HID), lambda i: (0, 0)),
        ],
        out_specs=pl.BlockSpec((_BE, HID), lambda i: (i, 0)),
        out_shape=jax.ShapeDtypeStruct((esize, HID), jnp.float32),
    )


def _final_body(agg_a_ref, agg_b_ref, node_ref, orig_ref, l1_ref, l2_ref,
                l3_ref, o1_ref, o2_ref, bo_ref, o_ref):
    agg = agg_a_ref[0] + agg_a_ref[1] + agg_b_ref[0] + agg_b_ref[1]
    n2 = jax.nn.relu(
        jnp.dot(agg, l1_ref[...], preferred_element_type=jnp.float32)
        + jnp.dot(node_ref[...], l2_ref[...], preferred_element_type=jnp.float32)
        + jnp.dot(orig_ref[...], l3_ref[...], preferred_element_type=jnp.float32))
    o_ref[...] = jax.nn.relu(
        jnp.dot(n2, o1_ref[...], preferred_element_type=jnp.float32)
        + jnp.dot(orig_ref[...], o2_ref[...], preferred_element_type=jnp.float32)
        + bo_ref[...])


_final = pl.pallas_call(
    _final_body,
    grid=(N // _BN,),
    in_specs=[
        pl.BlockSpec((NC, _BN, HID), lambda i: (0, i, 0)),
        pl.BlockSpec((NC, _BN, HID), lambda i: (0, i, 0)),
        pl.BlockSpec((_BN, HID), lambda i: (i, 0)),
        pl.BlockSpec((_BN, HID), lambda i: (i, 0)),
        pl.BlockSpec((HID, HID), lambda i: (0, 0)),
        pl.BlockSpec((HID, HID), lambda i: (0, 0)),
        pl.BlockSpec((HID, HID), lambda i: (0, 0)),
        pl.BlockSpec((HID, HID), lambda i: (0, 0)),
        pl.BlockSpec((HID, HID), lambda i: (0, 0)),
        pl.BlockSpec((1, HID), lambda i: (0, 0)),
    ],
    out_specs=pl.BlockSpec((_BN, HID), lambda i: (i, 0)),
    out_shape=jax.ShapeDtypeStruct((N, HID), jnp.float32),
)


_HALVES = ((EA, 0), (EB, EA // _BE))


def kernel(x, edge_index, edge_attr, W_i_atom, W_i_bond, W_h_atom, W_h_0,
           W_h_1, lr_W, W_o, b_o):
    src = edge_index[0]
    dst = edge_index[1]
    src_h = (src[:EA].reshape(NW, EA // NW // CHUNK, CHUNK),
             src[EA:].reshape(NW, EB // NW // CHUNK, CHUNK))
    dst_h = (dst[:EA].reshape(NW, EA // NW // CHUNK, CHUNK),
             dst[EA:].reshape(NW, EB // NW // CHUNK, CHUNK))
    zeros = jnp.zeros((N_PAD, HID), jnp.float32)

    xt = x.T
    eat = edge_attr.T
    node_origin = _node_init(xt, W_i_atom.T)
    edge_h = [_edge_init(es, off)(eat, W_i_bond.T) for es, off in _HALVES]

    a1 = W_h_atom[:, :HID].T
    a2 = W_h_atom[:, HID:].T
    node = node_origin
    for wd in (W_h_0, W_h_1):
        agg_a = _sc_scatter_kernel(EA)(edge_h[0], dst_h[0], zeros)
        agg_b = _sc_scatter_kernel(EB)(edge_h[1], dst_h[1], zeros)
        node = _node_mult(agg_a, agg_b, node)
        gath_a = _sc_gather_kernel(EA)(node, src_h[0])
        gath_b = _sc_gather_kernel(EB)(node, src_h[1])
        wdt = wd.T
        edge_h = [_edge_update(EA, 0)(gath_a, eat, a1, a2, wdt),
                  _edge_update(EB, EA // _BE)(gath_b, eat, a1, a2, wdt)]

    agg_a = _sc_scatter_kernel(EA)(edge_h[0], dst_h[0], zeros)
    agg_b = _sc_scatter_kernel(EB)(edge_h[1], dst_h[1], zeros)
    out = _final(
        agg_a, agg_b, node, node_origin,
        lr_W[:, :HID].T, lr_W[:, HID:2 * HID].T, lr_W[:, 2 * HID:].T,
        W_o[:, :HID].T, W_o[:, HID:].T, b_o.reshape(1, HID))
    return out
